# Initial kernel scaffold; baseline (speedup 1.0000x reference)
import functools

import jax
import jax.numpy as jnp
from jax import lax
from jax.experimental import pallas as pl
from jax.experimental.pallas import tpu as pltpu
from jax.experimental.pallas import tpu_sc as plsc

VOCAB = 1000
EMBED_DIM = 64
BATCH = 4096
SEQ = 200

N = BATCH * SEQ
NUM_CORES = 2
NUM_SUBCORES = 16
NW = NUM_CORES * NUM_SUBCORES
PER_W = N // NW
IDX_MINOR = 128
CHUNK = 512
IDX_ROWS = CHUNK // IDX_MINOR
NCHUNKS = PER_W // CHUNK

_mesh = plsc.VectorSubcoreMesh(core_axis_name="c", subcore_axis_name="s")


@functools.partial(
    pl.kernel,
    mesh=_mesh,
    out_type=jax.ShapeDtypeStruct((N, EMBED_DIM), jnp.float32),
    scratch_types=[
        pltpu.VMEM((IDX_ROWS, IDX_MINOR), jnp.int32),
        pltpu.VMEM((CHUNK, EMBED_DIM), jnp.float32),
        pltpu.SemaphoreType.DMA,
    ],
)
def _sc_gather(idx_hbm, table_hbm, out_hbm, idx_v, rows_v, sem):
    wid = lax.axis_index("s") * NUM_CORES + lax.axis_index("c")
    idx_row0 = wid * (PER_W // IDX_MINOR)
    out_row0 = wid * PER_W

    def body(ci, carry):
        pltpu.sync_copy(idx_hbm.at[pl.ds(idx_row0 + ci * IDX_ROWS, IDX_ROWS)],
                        idx_v)
        copies = [
            pltpu.async_copy(table_hbm.at[idx_v.at[j]],
                             rows_v.at[pl.ds(j * IDX_MINOR, IDX_MINOR)],
                             sem)
            for j in range(IDX_ROWS)
        ]
        for cp in copies:
            cp.wait()
        pltpu.sync_copy(rows_v,
                        out_hbm.at[pl.ds(out_row0 + ci * CHUNK, CHUNK)])
        return carry

    lax.fori_loop(0, NCHUNKS, body, 0)


def kernel(word_sequences, table):
    idx = word_sequences.reshape(N // IDX_MINOR, IDX_MINOR).astype(jnp.int32)
    out = _sc_gather(idx, table)
    return out.reshape(BATCH, SEQ, EMBED_DIM)


# speedup vs baseline: 3.5976x; 3.5976x over previous
import functools

import jax
import jax.numpy as jnp
from jax import lax
from jax.experimental import pallas as pl
from jax.experimental.pallas import tpu as pltpu
from jax.experimental.pallas import tpu_sc as plsc

VOCAB = 1000
EMBED_DIM = 64
BATCH = 4096
SEQ = 200

N = BATCH * SEQ
NUM_CORES = 2
NUM_SUBCORES = 16
NW = NUM_CORES * NUM_SUBCORES
PER_W = N // NW
IDX_MINOR = 128
CHUNK = 512
IDX_ROWS = CHUNK // IDX_MINOR
NCHUNKS = PER_W // CHUNK

_mesh = plsc.VectorSubcoreMesh(core_axis_name="c", subcore_axis_name="s")


@functools.partial(
    pl.kernel,
    mesh=_mesh,
    out_type=jax.ShapeDtypeStruct((N, EMBED_DIM), jnp.float32),
    scratch_types=[
        pltpu.VMEM((IDX_ROWS, IDX_MINOR), jnp.int32),
        pltpu.VMEM((CHUNK, EMBED_DIM), jnp.float32),
        pltpu.SemaphoreType.DMA,
    ],
    compiler_params=pltpu.CompilerParams(use_tc_tiling_on_sc=False),
)
def _sc_gather(idx_hbm, table_hbm, out_hbm, idx_v, rows_v, sem):
    wid = lax.axis_index("s") * NUM_CORES + lax.axis_index("c")
    idx_row0 = wid * (PER_W // IDX_MINOR)
    out_row0 = wid * PER_W

    def body(ci, carry):
        pltpu.sync_copy(idx_hbm.at[pl.ds(idx_row0 + ci * IDX_ROWS, IDX_ROWS)],
                        idx_v)
        copies = [
            pltpu.async_copy(table_hbm.at[idx_v.at[j]],
                             rows_v.at[pl.ds(j * IDX_MINOR, IDX_MINOR)],
                             sem)
            for j in range(IDX_ROWS)
        ]
        for cp in copies:
            cp.wait()
        pltpu.sync_copy(rows_v,
                        out_hbm.at[pl.ds(out_row0 + ci * CHUNK, CHUNK)])
        return carry

    lax.fori_loop(0, NCHUNKS, body, 0)


def kernel(word_sequences, table):
    idx = word_sequences.reshape(N // IDX_MINOR, IDX_MINOR).astype(jnp.int32)
    out = _sc_gather(idx, table)
    return out.reshape(BATCH, SEQ, EMBED_DIM)
